# baseline (speedup 1.0000x reference)
import functools

import jax
import jax.numpy as jnp
from jax import lax
from jax.experimental import pallas as pl
from jax.experimental.pallas import tpu as pltpu
from jax.experimental.pallas import tpu_sc as plsc

POOL = 16384
DFF = 3072
DFW = DFF // 2
DM = 768
NTOK = 8192
K = 8
NC, NS, LANES = 2, 16, 16
NW = NC * NS
TPW = NTOK // NW
GRP = TPW // LANES
OUT_T = 8
NBUF = 4
UNROLL = 2

_f32 = jnp.float32
_bf16 = jnp.bfloat16
_i32 = jnp.int32


def _sc_body(fp_hbm, idx_hbm, w_hbm, out_hbm,
             idx_v, w_v, rows_v, out_v, gsems, osems):
    wid = lax.axis_index("s") * NC + lax.axis_index("c")
    base = wid * TPW

    pltpu.sync_copy(idx_hbm.at[pl.ds(base * K, TPW * K)], idx_v)
    pltpu.sync_copy(w_hbm.at[:, pl.ds(base, TPW)], w_v)

    def softmax_g(g, carry):
        col = g * LANES
        wv = [w_v[k, pl.ds(col, LANES)] for k in range(K)]
        m = wv[0]
        for k in range(1, K):
            m = jnp.maximum(m, wv[k])
        e = [jnp.exp(v - m) for v in wv]
        s = e[0]
        for k in range(1, K):
            s = s + e[k]
        inv = 1.0 / s
        for k in range(K):
            w_v[k, pl.ds(col, LANES)] = e[k] * inv
        return carry
    lax.fori_loop(0, GRP, softmax_g, 0)

    def g_copy(t, b):
        return pltpu.make_async_copy(
            fp_hbm.at[idx_v.at[pl.ds(t * K, K)]], rows_v.at[b], gsems[b])

    def o_copy(row, ob):
        return pltpu.make_async_copy(
            out_v.at[ob], out_hbm.at[pl.ds(row, OUT_T)], osems[ob])

    for b in range(NBUF - 1):
        g_copy(b, b).start()

    def super_body(i, carry):
        wg = [w_v[k, pl.ds(i * LANES, LANES)] for k in range(K)]
        for j in range(16):
            t = i * 16 + j
            b = j % NBUF
            ob = j // 8
            if j == 0:
                @pl.when(i > 0)
                def _w0():
                    o_copy(base + (i - 1) * 16, 0).wait()
            if j == 8:
                @pl.when(i > 0)
                def _w1():
                    o_copy(base + (i - 1) * 16 + OUT_T, 1).wait()

            g_copy(t, b).wait()
            @pl.when(t + NBUF - 1 < TPW)
            def _g():
                g_copy(t + NBUF - 1, (j + NBUF - 1) % NBUF).start()

            spl = [jnp.broadcast_to(wg[k][j], (LANES,)) for k in range(K)]

            def chunk_body(c, carry2, _b=b, _ob=ob, _jj=j % 8, _spl=spl):
                for u in range(UNROLL):
                    off = (c * UNROLL + u) * LANES
                    acc_e = jnp.zeros((LANES,), _f32)
                    acc_o = jnp.zeros((LANES,), _f32)
                    for k in range(K):
                        pk = plsc.bitcast(
                            rows_v[_b, k, pl.ds(off, LANES)], _bf16)
                        ev, ov = plsc.unpack(pk, format=plsc.PackFormat.INTERLEAVED)
                        acc_e = acc_e + _spl[k] * ev
                        acc_o = acc_o + _spl[k] * ov
                    out_v[_ob, _jj, pl.ds(off, LANES)] = plsc.bitcast(
                        plsc.pack(acc_e, acc_o,
                                  format=plsc.PackFormat.INTERLEAVED), _i32)
                return carry2
            lax.fori_loop(0, DFW // (LANES * UNROLL), chunk_body, 0)

            if j == 7:
                o_copy(base + i * 16, 0).start()
            if j == 15:
                o_copy(base + i * 16 + OUT_T, 1).start()
        return carry

    lax.fori_loop(0, GRP, super_body, 0)

    o_copy(base + (GRP - 1) * 16, 0).wait()
    o_copy(base + (GRP - 1) * 16 + OUT_T, 1).wait()


_sc_combine = functools.partial(
    pl.kernel,
    out_type=jax.ShapeDtypeStruct((NTOK, DFW), _i32),
    mesh=plsc.VectorSubcoreMesh(
        core_axis_name="c", subcore_axis_name="s",
        num_cores=NC, num_subcores=NS),
    compiler_params=pltpu.CompilerParams(needs_layout_passes=False),
    scratch_types=[
        pltpu.VMEM((TPW * K,), _i32),
        pltpu.VMEM((K, TPW), _f32),
        pltpu.VMEM((NBUF, K, DFW), _i32),
        pltpu.VMEM((2, OUT_T, DFW), _i32),
        [pltpu.SemaphoreType.DMA] * NBUF,
        [pltpu.SemaphoreType.DMA] * 2,
    ],
)(_sc_body)


TBLK = 512


def _tc_body(x_ref, w_ref, b_ref, o_ref):
    x = x_ref[...].astype(_f32)
    a = 0.5 * x * (1.0 + lax.erf(x * (2.0 ** -0.5)))
    o_ref[...] = (jnp.dot(a.astype(_bf16), w_ref[...],
                          preferred_element_type=_f32)
                  + b_ref[...])


def _tc_gelu_matmul(x, wt, b2):
    return pl.pallas_call(
        _tc_body,
        grid=(NTOK // TBLK,),
        in_specs=[
            pl.BlockSpec((TBLK, DFF), lambda i: (i, 0)),
            pl.BlockSpec((DFF, DM), lambda i: (0, 0)),
            pl.BlockSpec((1, DM), lambda i: (0, 0)),
        ],
        out_specs=pl.BlockSpec((TBLK, DM), lambda i: (i, 0)),
        out_shape=jax.ShapeDtypeStruct((NTOK, DM), _f32),
    )(x, wt, b2)


def kernel(selected_indices, pattern_weights, firing_patterns, W2_w, W2_b):
    B, S, _ = selected_indices.shape
    idx = selected_indices.reshape(NTOK * K).astype(_i32)
    wT = pattern_weights.reshape(NTOK, K).T
    fp16 = firing_patterns.astype(_bf16)
    fpi = lax.bitcast_convert_type(
        fp16.reshape(POOL, DFW, 2), _i32)
    ci = _sc_combine(fpi, idx, wT)
    combined = lax.bitcast_convert_type(ci, _bf16).reshape(NTOK, DFF)
    out = _tc_gelu_matmul(combined, W2_w.T.astype(_bf16), W2_b.reshape(1, DM))
    return out.reshape(B, S, DM)


# speedup vs baseline: 3.0501x; 3.0501x over previous
import functools

import jax
import jax.numpy as jnp
import numpy as np
from jax import lax
from jax.experimental import pallas as pl
from jax.experimental.pallas import tpu as pltpu
from jax.experimental.pallas import tpu_sc as plsc

POOL = 16384
DFF = 3072
DFW = DFF // 2
DM = 768
NTOK = 8192
K = 8
NC, NS, LANES = 2, 16, 16
NW = NC * NS
TPW = NTOK // NW
GRP = TPW // LANES
OUT_T = 8
NBUF = 4
UNROLL = 2

_f32 = jnp.float32
_bf16 = jnp.bfloat16
_i32 = jnp.int32


PBLK = 512


def _pack_body(x_ref, o_ref):
    x = x_ref[...]
    lo = pltpu.bitcast(x[:, :DFW], _i32)
    hi = pltpu.bitcast(x[:, DFW:], _i32)

    def rne16(b):
        return lax.shift_right_logical(
            b + 0x7FFF + (lax.shift_right_logical(b, 16) & 1), 16)

    o_ref[...] = rne16(lo) | lax.shift_left(rne16(hi), 16)


def _tc_pack(fp):
    return pl.pallas_call(
        _pack_body,
        grid=(POOL // PBLK,),
        in_specs=[pl.BlockSpec((PBLK, DFF), lambda i: (i, 0))],
        out_specs=pl.BlockSpec((PBLK, DFW), lambda i: (i, 0)),
        out_shape=jax.ShapeDtypeStruct((POOL, DFW), _i32),
    )(fp)


def _take16(v, idx):
    return lax.gather(
        v, idx[:, None],
        lax.GatherDimensionNumbers(offset_dims=(), collapsed_slice_dims=(0,),
                                   start_index_map=(0,)),
        (1,), mode=lax.GatherScatterMode.PROMISE_IN_BOUNDS)


def _sc_body(fp_hbm, idx_hbm, w_hbm, out_hbm,
             idx_v, w_v, rows_v, out_v, gsems, osems):
    wid = lax.axis_index("s") * NC + lax.axis_index("c")
    base = wid * TPW

    pltpu.sync_copy(idx_hbm.at[pl.ds(base * K, TPW * K)], idx_v)
    pltpu.sync_copy(w_hbm.at[pl.ds(base * K, TPW * K)], w_v)

    lane = lax.iota(_i32, LANES)
    xor_idx = [lane ^ d for d in (1, 2, 4)]

    def softmax_step(g, carry):
        tw = w_v[pl.ds(g * LANES, LANES)]
        m = tw
        for x in xor_idx:
            m = jnp.maximum(m, _take16(m, x))
        e = jnp.exp(tw - m)
        s = e
        for x in xor_idx:
            s = s + _take16(s, x)
        w_v[pl.ds(g * LANES, LANES)] = e / s
        return carry
    lax.fori_loop(0, TPW * K // LANES, softmax_step, 0)

    def g_copy(t, b):
        return pltpu.make_async_copy(
            fp_hbm.at[idx_v.at[pl.ds(t * K, K)]], rows_v.at[b], gsems[b])

    def o_copy(row, ob):
        return pltpu.make_async_copy(
            out_v.at[ob], out_hbm.at[pl.ds(row, OUT_T)], osems[ob])

    for b in range(NBUF - 1):
        g_copy(b, b).start()

    def super_body(i, carry):
        for j in range(16):
            t = i * 16 + j
            b = j % NBUF
            ob = j // 8
            if j == 0:
                @pl.when(i > 0)
                def _w0():
                    o_copy(base + (i - 1) * 16, 0).wait()
            if j == 8:
                @pl.when(i > 0)
                def _w1():
                    o_copy(base + (i - 1) * 16 + OUT_T, 1).wait()

            g_copy(t, b).wait()
            @pl.when(t + NBUF - 1 < TPW)
            def _g():
                g_copy(t + NBUF - 1, (j + NBUF - 1) % NBUF).start()

            wchunk = w_v[pl.ds((i * 16 + j) // 2 * LANES, LANES)]
            spl = [jnp.broadcast_to(wchunk[(j % 2) * K + k], (LANES,))
                   for k in range(K)]

            def chunk_body(c, carry2, _b=b, _ob=ob, _jj=j % 8, _spl=spl):
                for u in range(UNROLL):
                    off = (c * UNROLL + u) * LANES
                    acc_a = jnp.zeros((LANES,), _f32)
                    acc_b = jnp.zeros((LANES,), _f32)
                    for k in range(K):
                        pk = plsc.bitcast(
                            rows_v[_b, k, pl.ds(off, LANES)], _bf16)
                        va, vb = plsc.unpack(
                            pk, format=plsc.PackFormat.INTERLEAVED)
                        acc_a = acc_a + _spl[k] * va
                        acc_b = acc_b + _spl[k] * vb
                    out_v[_ob, _jj, pl.ds(off, LANES)] = acc_a
                    out_v[_ob, _jj, pl.ds(DFW + off, LANES)] = acc_b
                return carry2
            lax.fori_loop(0, DFW // (LANES * UNROLL), chunk_body, 0)

            if j == 7:
                o_copy(base + i * 16, 0).start()
            if j == 15:
                o_copy(base + i * 16 + OUT_T, 1).start()
        return carry

    lax.fori_loop(0, GRP, super_body, 0)

    o_copy(base + (GRP - 1) * 16, 0).wait()
    o_copy(base + (GRP - 1) * 16 + OUT_T, 1).wait()


_sc_combine = functools.partial(
    pl.kernel,
    out_type=jax.ShapeDtypeStruct((NTOK, DFF), _f32),
    mesh=plsc.VectorSubcoreMesh(
        core_axis_name="c", subcore_axis_name="s",
        num_cores=NC, num_subcores=NS),
    compiler_params=pltpu.CompilerParams(needs_layout_passes=False),
    scratch_types=[
        pltpu.VMEM((TPW * K,), _i32),
        pltpu.VMEM((TPW * K,), _f32),
        pltpu.VMEM((NBUF, K, DFW), _i32),
        pltpu.VMEM((2, OUT_T, DFF), _f32),
        [pltpu.SemaphoreType.DMA] * NBUF,
        [pltpu.SemaphoreType.DMA] * 2,
    ],
)(_sc_body)


TBLK = 512


def _tc_body(x_ref, w_ref, b_ref, o_ref):
    x = x_ref[...]
    a = 0.5 * x * (1.0 + lax.erf(x * (2.0 ** -0.5)))
    o_ref[...] = lax.dot_general(
        a.astype(_bf16), w_ref[...].astype(_bf16),
        (((1,), (1,)), ((), ())),
        preferred_element_type=_f32) + b_ref[...]


def _tc_gelu_matmul(x, w2, b2):
    return pl.pallas_call(
        _tc_body,
        grid=(NTOK // TBLK,),
        in_specs=[
            pl.BlockSpec((TBLK, DFF), lambda i: (i, 0)),
            pl.BlockSpec((DM, DFF), lambda i: (0, 0)),
            pl.BlockSpec((1, DM), lambda i: (0, 0)),
        ],
        out_specs=pl.BlockSpec((TBLK, DM), lambda i: (i, 0)),
        out_shape=jax.ShapeDtypeStruct((NTOK, DM), _f32),
    )(x, w2, b2)


def kernel(selected_indices, pattern_weights, firing_patterns, W2_w, W2_b):
    B, S, _ = selected_indices.shape
    idx = selected_indices.reshape(NTOK * K).astype(_i32)
    wflat = pattern_weights.reshape(NTOK * K)
    fpi = _tc_pack(firing_patterns)
    combined = _sc_combine(fpi, idx, wflat)
    out = _tc_gelu_matmul(combined, W2_w, W2_b.reshape(1, DM))
    return out.reshape(B, S, DM)


# speedup vs baseline: 3.1332x; 1.0273x over previous
import functools

import jax
import jax.numpy as jnp
import numpy as np
from jax import lax
from jax.experimental import pallas as pl
from jax.experimental.pallas import tpu as pltpu
from jax.experimental.pallas import tpu_sc as plsc

POOL = 16384
DFF = 3072
DFW = DFF // 2
DM = 768
NTOK = 8192
K = 8
NC, NS, LANES = 2, 16, 16
NW = NC * NS
TPW = NTOK // NW
GRP = TPW // LANES
OUT_T = 8
NBUF = 4
UNROLL = 2

_f32 = jnp.float32
_bf16 = jnp.bfloat16
_i32 = jnp.int32


PBLK = 512


def _pack_body(x_ref, o_ref):
    x = x_ref[...]
    lo = pltpu.bitcast(x[:, :DFW], _i32)
    hi = pltpu.bitcast(x[:, DFW:], _i32)

    def rne16(b):
        return lax.shift_right_logical(
            b + 0x7FFF + (lax.shift_right_logical(b, 16) & 1), 16)

    o_ref[...] = rne16(lo) | lax.shift_left(rne16(hi), 16)


def _tc_pack(fp):
    return pl.pallas_call(
        _pack_body,
        grid=(POOL // PBLK,),
        in_specs=[pl.BlockSpec((PBLK, DFF), lambda i: (i, 0))],
        out_specs=pl.BlockSpec((PBLK, DFW), lambda i: (i, 0)),
        out_shape=jax.ShapeDtypeStruct((POOL, DFW), _i32),
    )(fp)


def _take16(v, idx):
    return lax.gather(
        v, idx[:, None],
        lax.GatherDimensionNumbers(offset_dims=(), collapsed_slice_dims=(0,),
                                   start_index_map=(0,)),
        (1,), mode=lax.GatherScatterMode.PROMISE_IN_BOUNDS)


def _sc_body(fp_hbm, idx_hbm, w_hbm, out_hbm,
             idx_v, w_v, rows_v, out_v, gsems, osems):
    wid = lax.axis_index("s") * NC + lax.axis_index("c")
    base = wid * TPW

    pltpu.sync_copy(idx_hbm.at[pl.ds(base * K, TPW * K)], idx_v)
    pltpu.sync_copy(w_hbm.at[pl.ds(base * K, TPW * K)], w_v)

    lane = lax.iota(_i32, LANES)
    xor_idx = [lane ^ d for d in (1, 2, 4)]

    def softmax_step(g, carry):
        tw = w_v[pl.ds(g * LANES, LANES)]
        m = tw
        for x in xor_idx:
            m = jnp.maximum(m, _take16(m, x))
        e = jnp.exp(tw - m)
        s = e
        for x in xor_idx:
            s = s + _take16(s, x)
        w_v[pl.ds(g * LANES, LANES)] = e / s
        return carry
    lax.fori_loop(0, TPW * K // LANES, softmax_step, 0)

    def g_copy(t, b):
        return pltpu.make_async_copy(
            fp_hbm.at[idx_v.at[pl.ds(t * K, K)]], rows_v.at[b], gsems[b])

    def o_copy(row, ob):
        return pltpu.make_async_copy(
            out_v.at[ob], out_hbm.at[pl.ds(row, OUT_T)], osems[ob])

    for b in range(NBUF - 1):
        g_copy(b, b).start()

    def super_body(i, carry):
        for j in range(16):
            t = i * 16 + j
            b = j % NBUF
            ob = j // 8
            if j == 0:
                @pl.when(i > 0)
                def _w0():
                    o_copy(base + (i - 1) * 16, 0).wait()
            if j == 8:
                @pl.when(i > 0)
                def _w1():
                    o_copy(base + (i - 1) * 16 + OUT_T, 1).wait()

            g_copy(t, b).wait()
            @pl.when(t + NBUF - 1 < TPW)
            def _g():
                g_copy(t + NBUF - 1, (j + NBUF - 1) % NBUF).start()

            wchunk = w_v[pl.ds((i * 16 + j) // 2 * LANES, LANES)]
            spl = [jnp.broadcast_to(wchunk[(j % 2) * K + k], (LANES,))
                   for k in range(K)]
            splb = [plsc.pack(s, s, format=plsc.PackFormat.INTERLEAVED)
                    for s in spl]

            def chunk_body(c, carry2, _b=b, _ob=ob, _jj=j % 8, _splb=splb):
                for u in range(UNROLL):
                    off = (c * UNROLL + u) * LANES
                    pk = plsc.bitcast(rows_v[_b, 0, pl.ds(off, LANES)], _bf16)
                    acc = _splb[0] * pk
                    for k in range(1, K):
                        pk = plsc.bitcast(
                            rows_v[_b, k, pl.ds(off, LANES)], _bf16)
                        acc = acc + _splb[k] * pk
                    va, vb = plsc.unpack(
                        acc, format=plsc.PackFormat.INTERLEAVED)
                    out_v[_ob, _jj, pl.ds(off, LANES)] = va
                    out_v[_ob, _jj, pl.ds(DFW + off, LANES)] = vb
                return carry2
            lax.fori_loop(0, DFW // (LANES * UNROLL), chunk_body, 0)

            if j == 7:
                o_copy(base + i * 16, 0).start()
            if j == 15:
                o_copy(base + i * 16 + OUT_T, 1).start()
        return carry

    lax.fori_loop(0, GRP, super_body, 0)

    o_copy(base + (GRP - 1) * 16, 0).wait()
    o_copy(base + (GRP - 1) * 16 + OUT_T, 1).wait()


_sc_combine = functools.partial(
    pl.kernel,
    out_type=jax.ShapeDtypeStruct((NTOK, DFF), _f32),
    mesh=plsc.VectorSubcoreMesh(
        core_axis_name="c", subcore_axis_name="s",
        num_cores=NC, num_subcores=NS),
    compiler_params=pltpu.CompilerParams(needs_layout_passes=False),
    scratch_types=[
        pltpu.VMEM((TPW * K,), _i32),
        pltpu.VMEM((TPW * K,), _f32),
        pltpu.VMEM((NBUF, K, DFW), _i32),
        pltpu.VMEM((2, OUT_T, DFF), _f32),
        [pltpu.SemaphoreType.DMA] * NBUF,
        [pltpu.SemaphoreType.DMA] * 2,
    ],
)(_sc_body)


TBLK = 512


def _tc_body(x_ref, w_ref, b_ref, o_ref):
    x = x_ref[...]
    a = 0.5 * x * (1.0 + lax.erf(x * (2.0 ** -0.5)))
    o_ref[...] = lax.dot_general(
        a.astype(_bf16), w_ref[...].astype(_bf16),
        (((1,), (1,)), ((), ())),
        preferred_element_type=_f32) + b_ref[...]


def _tc_gelu_matmul(x, w2, b2):
    return pl.pallas_call(
        _tc_body,
        grid=(NTOK // TBLK,),
        in_specs=[
            pl.BlockSpec((TBLK, DFF), lambda i: (i, 0)),
            pl.BlockSpec((DM, DFF), lambda i: (0, 0)),
            pl.BlockSpec((1, DM), lambda i: (0, 0)),
        ],
        out_specs=pl.BlockSpec((TBLK, DM), lambda i: (i, 0)),
        out_shape=jax.ShapeDtypeStruct((NTOK, DM), _f32),
    )(x, w2, b2)


def kernel(selected_indices, pattern_weights, firing_patterns, W2_w, W2_b):
    B, S, _ = selected_indices.shape
    idx = selected_indices.reshape(NTOK * K).astype(_i32)
    wflat = pattern_weights.reshape(NTOK * K)
    fpi = _tc_pack(firing_patterns)
    combined = _sc_combine(fpi, idx, wflat)
    out = _tc_gelu_matmul(combined, W2_w, W2_b.reshape(1, DM))
    return out.reshape(B, S, DM)


# speedup vs baseline: 3.3559x; 1.0711x over previous
import functools

import jax
import jax.numpy as jnp
import numpy as np
from jax import lax
from jax.experimental import pallas as pl
from jax.experimental.pallas import tpu as pltpu
from jax.experimental.pallas import tpu_sc as plsc

POOL = 16384
DFF = 3072
DFW = DFF // 2
DM = 768
NTOK = 8192
K = 8
NC, NS, LANES = 2, 16, 16
NW = NC * NS
TPW = NTOK // NW
GRP = TPW // LANES
OUT_T = 8
NBUF = 4
UNROLL = 2

_f32 = jnp.float32
_bf16 = jnp.bfloat16
_i32 = jnp.int32


PBLK = 512


def _pack_body(x_ref, o_ref):
    x = x_ref[...]
    lo = pltpu.bitcast(x[:, :DFW], _i32)
    hi = pltpu.bitcast(x[:, DFW:], _i32)

    def rne16(b):
        return lax.shift_right_logical(
            b + 0x7FFF + (lax.shift_right_logical(b, 16) & 1), 16)

    o_ref[...] = rne16(lo) | lax.shift_left(rne16(hi), 16)


def _tc_pack(fp):
    return pl.pallas_call(
        _pack_body,
        grid=(POOL // PBLK,),
        in_specs=[pl.BlockSpec((PBLK, DFF), lambda i: (i, 0))],
        out_specs=pl.BlockSpec((PBLK, DFW), lambda i: (i, 0)),
        out_shape=jax.ShapeDtypeStruct((POOL, DFW), _i32),
    )(fp)


def _take16(v, idx):
    return lax.gather(
        v, idx[:, None],
        lax.GatherDimensionNumbers(offset_dims=(), collapsed_slice_dims=(0,),
                                   start_index_map=(0,)),
        (1,), mode=lax.GatherScatterMode.PROMISE_IN_BOUNDS)


def _sc_body(fp_hbm, idx_hbm, w_hbm, out_hbm,
             idx_v, w_v, rows_v, out_v, gsems, osems):
    wid = lax.axis_index("s") * NC + lax.axis_index("c")
    base = wid * TPW

    pltpu.sync_copy(idx_hbm.at[pl.ds(base * K, TPW * K)], idx_v)
    pltpu.sync_copy(w_hbm.at[pl.ds(base * K, TPW * K)], w_v)

    lane = lax.iota(_i32, LANES)
    xor_idx = [lane ^ d for d in (1, 2, 4)]

    def softmax_step(g, carry):
        tw = w_v[pl.ds(g * LANES, LANES)]
        m = tw
        for x in xor_idx:
            m = jnp.maximum(m, _take16(m, x))
        e = jnp.exp(tw - m)
        s = e
        for x in xor_idx:
            s = s + _take16(s, x)
        w_v[pl.ds(g * LANES, LANES)] = e / s
        return carry
    lax.fori_loop(0, TPW * K // LANES, softmax_step, 0)

    def g_copy(t, b):
        return pltpu.make_async_copy(
            fp_hbm.at[idx_v.at[pl.ds(t * K, K)]], rows_v.at[b], gsems[b])

    def o_copy(row, ob):
        return pltpu.make_async_copy(
            out_v.at[ob], out_hbm.at[pl.ds(row, OUT_T)], osems[ob])

    for b in range(NBUF - 1):
        g_copy(b, b).start()

    def super_body(i, carry):
        for j in range(16):
            t = i * 16 + j
            b = j % NBUF
            ob = j // 8
            if j == 0:
                @pl.when(i > 0)
                def _w0():
                    o_copy(base + (i - 1) * 16, 0).wait()
            if j == 8:
                @pl.when(i > 0)
                def _w1():
                    o_copy(base + (i - 1) * 16 + OUT_T, 1).wait()

            g_copy(t, b).wait()
            @pl.when(t + NBUF - 1 < TPW)
            def _g():
                g_copy(t + NBUF - 1, (j + NBUF - 1) % NBUF).start()

            wchunk = w_v[pl.ds((i * 16 + j) // 2 * LANES, LANES)]
            spl = [jnp.broadcast_to(wchunk[(j % 2) * K + k], (LANES,))
                   for k in range(K)]
            splb = [plsc.pack(s, s, format=plsc.PackFormat.INTERLEAVED)
                    for s in spl]

            def chunk_body(c, carry2, _b=b, _ob=ob, _jj=j % 8, _splb=splb):
                for u in range(UNROLL):
                    off = (c * UNROLL + u) * LANES
                    pk = plsc.bitcast(rows_v[_b, 0, pl.ds(off, LANES)], _bf16)
                    acc = _splb[0] * pk
                    for k in range(1, K):
                        pk = plsc.bitcast(
                            rows_v[_b, k, pl.ds(off, LANES)], _bf16)
                        acc = acc + _splb[k] * pk
                    out_v[_ob, _jj, pl.ds(off, LANES)] = plsc.bitcast(
                        acc, _i32)
                return carry2
            lax.fori_loop(0, DFW // (LANES * UNROLL), chunk_body, 0)

            if j == 7:
                o_copy(base + i * 16, 0).start()
            if j == 15:
                o_copy(base + i * 16 + OUT_T, 1).start()
        return carry

    lax.fori_loop(0, GRP, super_body, 0)

    o_copy(base + (GRP - 1) * 16, 0).wait()
    o_copy(base + (GRP - 1) * 16 + OUT_T, 1).wait()


_sc_combine = functools.partial(
    pl.kernel,
    out_type=jax.ShapeDtypeStruct((NTOK, DFW), _i32),
    mesh=plsc.VectorSubcoreMesh(
        core_axis_name="c", subcore_axis_name="s",
        num_cores=NC, num_subcores=NS),
    compiler_params=pltpu.CompilerParams(needs_layout_passes=False),
    scratch_types=[
        pltpu.VMEM((TPW * K,), _i32),
        pltpu.VMEM((TPW * K,), _f32),
        pltpu.VMEM((NBUF, K, DFW), _i32),
        pltpu.VMEM((2, OUT_T, DFW), _i32),
        [pltpu.SemaphoreType.DMA] * NBUF,
        [pltpu.SemaphoreType.DMA] * 2,
    ],
)(_sc_body)


TBLK = 512


def _tc_body(x_ref, w_ref, b_ref, o_ref):
    xw = x_ref[...]
    lo = pltpu.bitcast(lax.shift_left(xw, 16), _f32)
    hi = pltpu.bitcast(xw & jnp.int32(-65536), _f32)
    w = w_ref[...]

    def gelu(x):
        return 0.5 * x * (1.0 + lax.erf(x * (2.0 ** -0.5)))

    nt = (((1,), (1,)), ((), ()))
    o_ref[...] = (
        lax.dot_general(gelu(lo).astype(_bf16), w[:, :DFW].astype(_bf16),
                        nt, preferred_element_type=_f32)
        + lax.dot_general(gelu(hi).astype(_bf16), w[:, DFW:].astype(_bf16),
                          nt, preferred_element_type=_f32)
        + b_ref[...])


def _tc_gelu_matmul(x, w2, b2):
    return pl.pallas_call(
        _tc_body,
        grid=(NTOK // TBLK,),
        in_specs=[
            pl.BlockSpec((TBLK, DFW), lambda i: (i, 0)),
            pl.BlockSpec((DM, DFF), lambda i: (0, 0)),
            pl.BlockSpec((1, DM), lambda i: (0, 0)),
        ],
        out_specs=pl.BlockSpec((TBLK, DM), lambda i: (i, 0)),
        out_shape=jax.ShapeDtypeStruct((NTOK, DM), _f32),
    )(x, w2, b2)


def kernel(selected_indices, pattern_weights, firing_patterns, W2_w, W2_b):
    B, S, _ = selected_indices.shape
    idx = selected_indices.reshape(NTOK * K).astype(_i32)
    wflat = pattern_weights.reshape(NTOK * K)
    fpi = _tc_pack(firing_patterns)
    combined = _sc_combine(fpi, idx, wflat)
    out = _tc_gelu_matmul(combined, W2_w, W2_b.reshape(1, DM))
    return out.reshape(B, S, DM)


# speedup vs baseline: 3.3841x; 1.0084x over previous
import functools

import jax
import jax.numpy as jnp
import numpy as np
from jax import lax
from jax.experimental import pallas as pl
from jax.experimental.pallas import tpu as pltpu
from jax.experimental.pallas import tpu_sc as plsc

POOL = 16384
DFF = 3072
DFW = DFF // 2
DM = 768
NTOK = 8192
K = 8
NC, NS, LANES = 2, 16, 16
NW = NC * NS
CH = 2
TPC = NTOK // CH
TPW = TPC // NW
GRP = TPW // LANES
OUT_T = 8
NBUF = 4
UNROLL = 2

_f32 = jnp.float32
_bf16 = jnp.bfloat16
_i32 = jnp.int32


PBLK = 512


def _pack_body(x_ref, o_ref):
    x = x_ref[...]
    lo = pltpu.bitcast(x[:, :DFW], _i32)
    hi = pltpu.bitcast(x[:, DFW:], _i32)

    def rne16(b):
        return lax.shift_right_logical(
            b + 0x7FFF + (lax.shift_right_logical(b, 16) & 1), 16)

    o_ref[...] = rne16(lo) | lax.shift_left(rne16(hi), 16)


def _tc_pack(fp):
    return pl.pallas_call(
        _pack_body,
        grid=(POOL // PBLK,),
        in_specs=[pl.BlockSpec((PBLK, DFF), lambda i: (i, 0))],
        out_specs=pl.BlockSpec((PBLK, DFW), lambda i: (i, 0)),
        out_shape=jax.ShapeDtypeStruct((POOL, DFW), _i32),
    )(fp)


def _take16(v, idx):
    return lax.gather(
        v, idx[:, None],
        lax.GatherDimensionNumbers(offset_dims=(), collapsed_slice_dims=(0,),
                                   start_index_map=(0,)),
        (1,), mode=lax.GatherScatterMode.PROMISE_IN_BOUNDS)


def _sc_body(chunk, fp_hbm, idx_hbm, w_hbm, out_hbm,
             idx_v, w_v, rows_v, out_v, gsems, osems):
    wid = lax.axis_index("s") * NC + lax.axis_index("c")
    base = wid * TPW
    gbase = chunk * TPC + base

    pltpu.sync_copy(idx_hbm.at[pl.ds(gbase * K, TPW * K)], idx_v)
    pltpu.sync_copy(w_hbm.at[pl.ds(gbase * K, TPW * K)], w_v)

    lane = lax.iota(_i32, LANES)
    xor_idx = [lane ^ d for d in (1, 2, 4)]

    def softmax_step(g, carry):
        tw = w_v[pl.ds(g * LANES, LANES)]
        m = tw
        for x in xor_idx:
            m = jnp.maximum(m, _take16(m, x))
        e = jnp.exp(tw - m)
        s = e
        for x in xor_idx:
            s = s + _take16(s, x)
        w_v[pl.ds(g * LANES, LANES)] = e / s
        return carry
    lax.fori_loop(0, TPW * K // LANES, softmax_step, 0)

    def g_copy(t, b):
        return pltpu.make_async_copy(
            fp_hbm.at[idx_v.at[pl.ds(t * K, K)]], rows_v.at[b], gsems[b])

    def o_copy(row, ob):
        return pltpu.make_async_copy(
            out_v.at[ob], out_hbm.at[pl.ds(row, OUT_T)], osems[ob])

    for b in range(NBUF - 1):
        g_copy(b, b).start()

    def super_body(i, carry):
        for j in range(16):
            t = i * 16 + j
            b = j % NBUF
            ob = j // 8
            if j == 0:
                @pl.when(i > 0)
                def _w0():
                    o_copy(base + (i - 1) * 16, 0).wait()
            if j == 8:
                @pl.when(i > 0)
                def _w1():
                    o_copy(base + (i - 1) * 16 + OUT_T, 1).wait()

            g_copy(t, b).wait()
            @pl.when(t + NBUF - 1 < TPW)
            def _g():
                g_copy(t + NBUF - 1, (j + NBUF - 1) % NBUF).start()

            wchunk = w_v[pl.ds((i * 16 + j) // 2 * LANES, LANES)]
            spl = [jnp.broadcast_to(wchunk[(j % 2) * K + k], (LANES,))
                   for k in range(K)]
            splb = [plsc.pack(s, s, format=plsc.PackFormat.INTERLEAVED)
                    for s in spl]

            def chunk_body(c, carry2, _b=b, _ob=ob, _jj=j % 8, _splb=splb):
                for u in range(UNROLL):
                    off = (c * UNROLL + u) * LANES
                    pk = plsc.bitcast(rows_v[_b, 0, pl.ds(off, LANES)], _bf16)
                    acc = _splb[0] * pk
                    for k in range(1, K):
                        pk = plsc.bitcast(
                            rows_v[_b, k, pl.ds(off, LANES)], _bf16)
                        acc = acc + _splb[k] * pk
                    out_v[_ob, _jj, pl.ds(off, LANES)] = plsc.bitcast(
                        acc, _i32)
                return carry2
            lax.fori_loop(0, DFW // (LANES * UNROLL), chunk_body, 0)

            if j == 7:
                o_copy(base + i * 16, 0).start()
            if j == 15:
                o_copy(base + i * 16 + OUT_T, 1).start()
        return carry

    lax.fori_loop(0, GRP, super_body, 0)

    o_copy(base + (GRP - 1) * 16, 0).wait()
    o_copy(base + (GRP - 1) * 16 + OUT_T, 1).wait()


_sc_combine = [
    functools.partial(
        pl.kernel,
        out_type=jax.ShapeDtypeStruct((TPC, DFW), _i32),
        mesh=plsc.VectorSubcoreMesh(
            core_axis_name="c", subcore_axis_name="s",
            num_cores=NC, num_subcores=NS),
        compiler_params=pltpu.CompilerParams(needs_layout_passes=False),
        scratch_types=[
            pltpu.VMEM((TPW * K,), _i32),
            pltpu.VMEM((TPW * K,), _f32),
            pltpu.VMEM((NBUF, K, DFW), _i32),
            pltpu.VMEM((2, OUT_T, DFW), _i32),
            [pltpu.SemaphoreType.DMA] * NBUF,
            [pltpu.SemaphoreType.DMA] * 2,
        ],
    )(functools.partial(_sc_body, c))
    for c in range(CH)
]


TBLK = 512


def _tc_body(x_ref, w_ref, b_ref, o_ref):
    xw = x_ref[...]
    lo = pltpu.bitcast(lax.shift_left(xw, 16), _f32)
    hi = pltpu.bitcast(xw & jnp.int32(-65536), _f32)
    w = w_ref[...]

    def gelu(x):
        return 0.5 * x * (1.0 + lax.erf(x * (2.0 ** -0.5)))

    nt = (((1,), (1,)), ((), ()))
    o_ref[...] = (
        lax.dot_general(gelu(lo).astype(_bf16), w[:, :DFW].astype(_bf16),
                        nt, preferred_element_type=_f32)
        + lax.dot_general(gelu(hi).astype(_bf16), w[:, DFW:].astype(_bf16),
                          nt, preferred_element_type=_f32)
        + b_ref[...])


def _tc_gelu_matmul(x, w2, b2, chunk, prev=None):
    blk0 = chunk * (TPC // TBLK)
    args = [x, w2, b2]
    in_specs = [
        pl.BlockSpec((TBLK, DFW), lambda i: (i, 0)),
        pl.BlockSpec((DM, DFF), lambda i: (0, 0)),
        pl.BlockSpec((1, DM), lambda i: (0, 0)),
    ]
    kwargs = {}
    if prev is not None:
        args.append(prev)
        in_specs.append(pl.BlockSpec((TBLK, DM), lambda i: (i, 0)))
        kwargs["input_output_aliases"] = {3: 0}
    return pl.pallas_call(
        lambda *refs: _tc_body(*refs[:3], refs[-1]),
        grid=(TPC // TBLK,),
        in_specs=in_specs,
        out_specs=pl.BlockSpec((TBLK, DM), lambda i, _b=blk0: (i + _b, 0)),
        out_shape=jax.ShapeDtypeStruct((NTOK, DM), _f32),
        **kwargs,
    )(*args)


def kernel(selected_indices, pattern_weights, firing_patterns, W2_w, W2_b):
    B, S, _ = selected_indices.shape
    idx = selected_indices.reshape(NTOK * K).astype(_i32)
    wflat = pattern_weights.reshape(NTOK * K)
    fpi = _tc_pack(firing_patterns)
    b2 = W2_b.reshape(1, DM)
    combined = [_sc_combine[c](fpi, idx, wflat) for c in range(CH)]
    out = None
    for c in range(CH):
        out = _tc_gelu_matmul(combined[c], W2_w, b2, c, prev=out)
    return out.reshape(B, S, DM)
